# baseline (speedup 1.0000x reference)
import functools

import jax
import jax.numpy as jnp
from jax import lax
from jax.experimental import pallas as pl
from jax.experimental.pallas import tpu as pltpu
from jax.experimental.pallas import tpu_sc as plsc

N = 10000
D = 128
E = 320000

NC = 2
NS = 16
L = 16

CHUNK = 128
PCH = 80
EPAD = NC * NS * PCH * CHUNK
TRASH = N
NPAD = 10240
HALFN = NPAD // 2
ACCR = HALFN + CHUNK
CAP = 5632
CCH = CAP // CHUNK
ROWS_PER_TILE = NPAD // NS

_mesh = plsc.VectorSubcoreMesh(core_axis_name="c", subcore_axis_name="s")


@functools.partial(
    pl.kernel,
    out_type=(
        jax.ShapeDtypeStruct((NC, NPAD), jnp.float32),
        jax.ShapeDtypeStruct((NC, NC * NS, CAP), jnp.int32),
        jax.ShapeDtypeStruct((NC, NC * NS, CAP), jnp.int32),
    ),
    mesh=_mesh,
    compiler_params=pltpu.CompilerParams(use_tc_tiling_on_sc=False,
                                         needs_layout_passes=False),
    scratch_types=[
        pltpu.VMEM((PCH, CHUNK), jnp.int32),
        pltpu.VMEM((PCH, CHUNK), jnp.int32),
        pltpu.VMEM((CAP,), jnp.int32),
        pltpu.VMEM((CAP,), jnp.int32),
        pltpu.VMEM((CAP,), jnp.int32),
        pltpu.VMEM((CAP,), jnp.int32),
        pltpu.VMEM((CHUNK,), jnp.float32),
        pltpu.VMEM((ROWS_PER_TILE,), jnp.float32),
        pltpu.VMEM_SHARED((NPAD,), jnp.float32),
    ],
)
def _prep_kernel(src_hbm, dst_hbm, degp_hbm, srcl_hbm, dstl_hbm,
                 src_v, dst_v, sa_v, da_v, sb_v, db_v, ones_v, dbuf_v, deg_sh):
    c = lax.axis_index("c")
    s = lax.axis_index("s")
    pid = c * NS + s
    pltpu.sync_copy(src_hbm.at[c, s], src_v)
    pltpu.sync_copy(dst_hbm.at[c, s], dst_v)

    @pl.loop(0, CAP // L)
    def _fill(i):
        sl = pl.ds(i * L, L)
        sa_v[sl] = jnp.zeros((L,), jnp.int32)
        da_v[sl] = jnp.full((L,), HALFN, jnp.int32)
        sb_v[sl] = jnp.zeros((L,), jnp.int32)
        db_v[sl] = jnp.full((L,), TRASH - HALFN, jnp.int32)

    @pl.loop(0, ROWS_PER_TILE // L)
    def _zero(i):
        dbuf_v[pl.ds(i * L, L)] = jnp.zeros((L,), jnp.float32)

    for k in range(CHUNK // L):
        ones_v[pl.ds(k * L, L)] = jnp.ones((L,), jnp.float32)
    pltpu.sync_copy(dbuf_v, deg_sh.at[pl.ds(s * ROWS_PER_TILE, ROWS_PER_TILE)])
    plsc.subcore_barrier()

    @pl.loop(0, PCH)
    def _acc(j):
        pltpu.sync_copy(ones_v, deg_sh.at[dst_v.at[j]], add=True)

    def _part(j, o):
        oa, ob = o
        for v in range(CHUNK // L):
            sv = src_v[j, pl.ds(v * L, L)]
            dv = dst_v[j, pl.ds(v * L, L)]
            in_a = dv < HALFN
            na = plsc.all_reduce_population_count(in_a)[0]
            plsc.store_compressed(sa_v.at[pl.ds(oa, L)], sv, mask=in_a)
            plsc.store_compressed(da_v.at[pl.ds(oa, L)], dv, mask=in_a)
            in_b = jnp.logical_not(in_a)
            plsc.store_compressed(sb_v.at[pl.ds(ob, L)], sv, mask=in_b)
            plsc.store_compressed(db_v.at[pl.ds(ob, L)], dv - HALFN, mask=in_b)
            oa = jnp.minimum(oa + na, CAP - L)
            ob = jnp.minimum(ob + (L - na), CAP - L)
        return oa, ob

    lax.fori_loop(0, PCH, _part, (jnp.int32(0), jnp.int32(0)))

    pltpu.sync_copy(sa_v, srcl_hbm.at[0, pid])
    pltpu.sync_copy(da_v, dstl_hbm.at[0, pid])
    pltpu.sync_copy(sb_v, srcl_hbm.at[1, pid])
    pltpu.sync_copy(db_v, dstl_hbm.at[1, pid])

    plsc.subcore_barrier()
    pltpu.sync_copy(deg_sh.at[pl.ds(s * ROWS_PER_TILE, ROWS_PER_TILE)], dbuf_v)
    pltpu.sync_copy(dbuf_v, degp_hbm.at[c, pl.ds(s * ROWS_PER_TILE, ROWS_PER_TILE)])


@functools.partial(
    pl.kernel,
    out_type=jax.ShapeDtypeStruct((NC, HALFN, D), jnp.float32),
    mesh=_mesh,
    compiler_params=pltpu.CompilerParams(use_tc_tiling_on_sc=False),
    scratch_types=[
        pltpu.VMEM((2 * CCH, CHUNK), jnp.int32),
        pltpu.VMEM((2 * CCH, CHUNK), jnp.int32),
        pltpu.VMEM((2, CHUNK, D), jnp.float32),
        pltpu.VMEM_SHARED((ACCR, D), jnp.float32),
    ] + [pltpu.SemaphoreType.DMA] * 2,
)
def _agg_kernel(hs_hbm, srcl_hbm, dstl_hbm, accp_hbm,
                src_v, dst_v, rows_v, acc_sh, *gs):
    c = lax.axis_index("c")
    s = lax.axis_index("s")
    pltpu.sync_copy(srcl_hbm.at[c, 2 * s], src_v.at[pl.ds(0, CCH)])
    pltpu.sync_copy(srcl_hbm.at[c, 2 * s + 1], src_v.at[pl.ds(CCH, CCH)])
    pltpu.sync_copy(dstl_hbm.at[c, 2 * s], dst_v.at[pl.ds(0, CCH)])
    pltpu.sync_copy(dstl_hbm.at[c, 2 * s + 1], dst_v.at[pl.ds(CCH, CCH)])

    @pl.loop(0, CHUNK)
    def _zero(r):
        for k in range(D // L):
            rows_v[0, r, pl.ds(k * L, L)] = jnp.zeros((L,), jnp.float32)

    for t in range(3):
        blk = s + NS * t

        @pl.when(blk < ACCR // CHUNK)
        def _z(blk=blk):
            pltpu.sync_copy(rows_v.at[0], acc_sh.at[pl.ds(blk * CHUNK, CHUNK)])

    plsc.subcore_barrier()

    NCH = 2 * CCH
    pltpu.async_copy(hs_hbm.at[src_v.at[0]], rows_v.at[0], gs[0]).wait()

    @pl.loop(0, NCH - 1)
    def _body(j):
        nxt = pltpu.async_copy(hs_hbm.at[src_v.at[j + 1]],
                               rows_v.at[(j + 1) % 2], gs[1])
        pltpu.sync_copy(rows_v.at[j % 2], acc_sh.at[dst_v.at[j]], add=True)
        nxt.wait()

    pltpu.sync_copy(rows_v.at[(NCH - 1) % 2],
                    acc_sh.at[dst_v.at[NCH - 1]], add=True)

    plsc.subcore_barrier()
    for t in range(3):
        blk = s + NS * t

        @pl.when(blk < HALFN // CHUNK)
        def _w(blk=blk):
            pltpu.sync_copy(acc_sh.at[pl.ds(blk * CHUNK, CHUNK)], rows_v.at[0])
            pltpu.sync_copy(rows_v.at[0], accp_hbm.at[c, pl.ds(blk * CHUNK, CHUNK)])


_BR = 512
_GRID = NPAD // _BR


def _tc_pre_body(x_ref, w_ref, d0_ref, d1_ref, hs_ref):
    dis = lax.rsqrt(d0_ref[...] + d1_ref[...] + 1.0)
    hs_ref[...] = jnp.dot(x_ref[...], w_ref[...],
                          preferred_element_type=jnp.float32) * dis


def _tc_mid_body(p_ref, h_ref, d0_ref, d1_ref, w_ref, bi_ref, hs_ref):
    dis = lax.rsqrt(d0_ref[...] + d1_ref[...] + 1.0)
    a = jnp.maximum(dis * (p_ref[...] + h_ref[...]) + bi_ref[...], 0.0)
    hs_ref[...] = jnp.dot(a, w_ref[...],
                          preferred_element_type=jnp.float32) * dis


def _tc_fin_body(p_ref, h_ref, d0_ref, d1_ref, w_ref, bi_ref, bo_ref, out_ref):
    dis = lax.rsqrt(d0_ref[...] + d1_ref[...] + 1.0)
    a = jnp.maximum(dis * (p_ref[...] + h_ref[...]) + bi_ref[...], 0.0)
    out_ref[...] = jnp.dot(a, w_ref[...],
                           preferred_element_type=jnp.float32) + bo_ref[...]


_row_spec = pl.BlockSpec((_BR, D), lambda i: (i, 0))
_deg_spec = pl.BlockSpec((_BR, 1), lambda i: (i, 0))
_w_spec = pl.BlockSpec((D, D), lambda i: (0, 0))
_b_spec = pl.BlockSpec((1, D), lambda i: (0, 0))

_row_out = jax.ShapeDtypeStruct((NPAD, D), jnp.float32)

_tc_pre = pl.pallas_call(
    _tc_pre_body,
    grid=(_GRID,),
    in_specs=[_row_spec, _w_spec, _deg_spec, _deg_spec],
    out_specs=_row_spec,
    out_shape=_row_out,
)

_tc_mid = pl.pallas_call(
    _tc_mid_body,
    grid=(_GRID,),
    in_specs=[_row_spec, _row_spec, _deg_spec, _deg_spec, _w_spec, _b_spec],
    out_specs=_row_spec,
    out_shape=_row_out,
)

_tc_fin = pl.pallas_call(
    _tc_fin_body,
    grid=(_GRID,),
    in_specs=[_row_spec, _row_spec, _deg_spec, _deg_spec, _w_spec, _b_spec,
              _b_spec],
    out_specs=_row_spec,
    out_shape=_row_out,
)


def kernel(x, edge_index, W1, b1, W2, b2, Wf, bf):
    src = edge_index[0]
    dst = edge_index[1]
    npad_e = EPAD - E
    src_t = jnp.concatenate(
        [src, jnp.zeros((npad_e,), jnp.int32)]).reshape(NC, NS, PCH, CHUNK)
    dst_t = jnp.concatenate(
        [dst, jnp.full((npad_e,), TRASH, jnp.int32)]).reshape(NC, NS, PCH, CHUNK)

    x_pad = jnp.concatenate([x, jnp.zeros((NPAD - N, D), jnp.float32)], axis=0)

    degp, srcl, dstl = _prep_kernel(src_t, dst_t)
    srcl = srcl.reshape(NC, NC * NS, CCH, CHUNK)
    dstl = dstl.reshape(NC, NC * NS, CCH, CHUNK)
    d0 = degp[0].reshape(NPAD, 1)
    d1 = degp[1].reshape(NPAD, 1)

    b1r = b1.reshape(1, D)
    b2r = b2.reshape(1, D)
    bfr = bf.reshape(1, D)

    hs1 = _tc_pre(x_pad, W1, d0, d1)
    acc1 = _agg_kernel(hs1, srcl, dstl).reshape(NPAD, D)
    hs2 = _tc_mid(acc1, hs1, d0, d1, W2, b1r)
    acc2 = _agg_kernel(hs2, srcl, dstl).reshape(NPAD, D)
    out = _tc_fin(acc2, hs2, d0, d1, Wf, b2r, bfr)
    return out[:N]


# speedup vs baseline: 3.9671x; 3.9671x over previous
import functools

import jax
import jax.numpy as jnp
from jax import lax
from jax.experimental import pallas as pl
from jax.experimental.pallas import tpu as pltpu
from jax.experimental.pallas import tpu_sc as plsc

N = 10000
D = 128
HALF = D // 2
E = 320000

NC = 2
NS = 16

CHUNK = 128
CHUNKS = 160
NB = 2
DCH = 80
EPAD = NS * CHUNKS * CHUNK
TRASH = N
NPAD = 10240
ROWS_PER_TILE = NPAD // NS
RB = ROWS_PER_TILE // CHUNK

_mesh = plsc.VectorSubcoreMesh(core_axis_name="c", subcore_axis_name="s")


@functools.partial(
    pl.kernel,
    out_type=jax.ShapeDtypeStruct((NC, NPAD), jnp.float32),
    mesh=_mesh,
    scratch_types=[
        pltpu.VMEM((DCH, CHUNK), jnp.int32),
        pltpu.VMEM((CHUNK,), jnp.float32),
        pltpu.VMEM((ROWS_PER_TILE,), jnp.float32),
        pltpu.VMEM_SHARED((NPAD,), jnp.float32),
    ],
)
def _deg_kernel(dst_hbm, degp_hbm, dst_v, ones_v, dbuf_v, deg_sh):
    c = lax.axis_index("c")
    s = lax.axis_index("s")
    pltpu.sync_copy(dst_hbm.at[c, s], dst_v)

    @pl.loop(0, ROWS_PER_TILE // 16)
    def _zero(i):
        dbuf_v[pl.ds(i * 16, 16)] = jnp.zeros((16,), jnp.float32)

    for k in range(CHUNK // 16):
        ones_v[pl.ds(k * 16, 16)] = jnp.ones((16,), jnp.float32)
    pltpu.sync_copy(dbuf_v, deg_sh.at[pl.ds(s * ROWS_PER_TILE, ROWS_PER_TILE)])
    plsc.subcore_barrier()

    @pl.loop(0, DCH)
    def _acc(j):
        pltpu.sync_copy(ones_v, deg_sh.at[dst_v.at[j]], add=True)

    plsc.subcore_barrier()
    pltpu.sync_copy(deg_sh.at[pl.ds(s * ROWS_PER_TILE, ROWS_PER_TILE)], dbuf_v)
    pltpu.sync_copy(dbuf_v, degp_hbm.at[c, pl.ds(s * ROWS_PER_TILE, ROWS_PER_TILE)])


@functools.partial(
    pl.kernel,
    out_type=jax.ShapeDtypeStruct((NC, NPAD, HALF), jnp.float32),
    mesh=_mesh,
    compiler_params=pltpu.CompilerParams(use_tc_tiling_on_sc=False),
    scratch_types=[
        pltpu.VMEM((CHUNKS, CHUNK), jnp.int32),
        pltpu.VMEM((CHUNKS, CHUNK), jnp.int32),
        pltpu.VMEM((NB, CHUNK, HALF), jnp.float32),
        pltpu.VMEM_SHARED((NPAD, HALF), jnp.float32),
    ] + [pltpu.SemaphoreType.DMA] * NB,
)
def _agg_kernel(hs0_hbm, hs1_hbm, src_hbm, dst_hbm, accp_hbm,
                src_v, dst_v, rows_v, acc_sh, *sems):
    gs = sems
    c = lax.axis_index("c")
    s = lax.axis_index("s")
    pltpu.sync_copy(src_hbm.at[s], src_v)
    pltpu.sync_copy(dst_hbm.at[s], dst_v)

    @pl.loop(0, CHUNK)
    def _zero(r):
        for k in range(HALF // 16):
            rows_v[0, r, pl.ds(k * 16, 16)] = jnp.zeros((16,), jnp.float32)

    for t in range(RB):
        pltpu.sync_copy(
            rows_v.at[0], acc_sh.at[pl.ds(s * ROWS_PER_TILE + t * CHUNK, CHUNK)])
    plsc.subcore_barrier()

    def run(hs_hbm):
        pltpu.async_copy(hs_hbm.at[src_v.at[0]], rows_v.at[0], gs[0]).wait()

        @pl.loop(0, CHUNKS - 1)
        def _body(j):
            nxt = pltpu.async_copy(hs_hbm.at[src_v.at[j + 1]],
                                   rows_v.at[(j + 1) % 2], gs[1])
            pltpu.sync_copy(rows_v.at[j % 2], acc_sh.at[dst_v.at[j]], add=True)
            nxt.wait()

        pltpu.sync_copy(rows_v.at[(CHUNKS - 1) % 2],
                        acc_sh.at[dst_v.at[CHUNKS - 1]], add=True)

    @pl.when(c == 0)
    def _c0():
        run(hs0_hbm)

    @pl.when(c == 1)
    def _c1():
        run(hs1_hbm)

    plsc.subcore_barrier()
    for t in range(RB):
        row0 = s * ROWS_PER_TILE + t * CHUNK
        pltpu.sync_copy(acc_sh.at[pl.ds(row0, CHUNK)], rows_v.at[0])
        pltpu.sync_copy(rows_v.at[0], accp_hbm.at[c, pl.ds(row0, CHUNK)])


_BR = 512
_GRID = NPAD // _BR


def _tc_pre_body(x_ref, w_ref, d0_ref, d1_ref, hs0_ref, hs1_ref):
    dis = lax.rsqrt(d0_ref[...] + d1_ref[...] + 1.0)
    hs = jnp.dot(x_ref[...], w_ref[...],
                 preferred_element_type=jnp.float32) * dis
    hs0_ref[...] = hs[:, :HALF]
    hs1_ref[...] = hs[:, HALF:]


def _tc_mid_body(p0_ref, p1_ref, h0_ref, h1_ref, d0_ref, d1_ref, w_ref,
                 bi_ref, hs0_ref, hs1_ref):
    dis = lax.rsqrt(d0_ref[...] + d1_ref[...] + 1.0)
    a = jnp.concatenate(
        [p0_ref[...] + h0_ref[...],
         p1_ref[...] + h1_ref[...]],
        axis=1)
    a = jnp.maximum(dis * a + bi_ref[...], 0.0)
    hs = jnp.dot(a, w_ref[...], preferred_element_type=jnp.float32) * dis
    hs0_ref[...] = hs[:, :HALF]
    hs1_ref[...] = hs[:, HALF:]


def _tc_fin_body(p0_ref, p1_ref, h0_ref, h1_ref, d0_ref, d1_ref, w_ref,
                 bi_ref, bo_ref, out_ref):
    dis = lax.rsqrt(d0_ref[...] + d1_ref[...] + 1.0)
    a = jnp.concatenate(
        [p0_ref[...] + h0_ref[...],
         p1_ref[...] + h1_ref[...]],
        axis=1)
    a = jnp.maximum(dis * a + bi_ref[...], 0.0)
    out_ref[...] = jnp.dot(a, w_ref[...],
                           preferred_element_type=jnp.float32) + bo_ref[...]


_row_spec = pl.BlockSpec((_BR, D), lambda i: (i, 0))
_half_spec = pl.BlockSpec((_BR, HALF), lambda i: (i, 0))
_deg_spec = pl.BlockSpec((_BR, 1), lambda i: (i, 0))
_w_spec = pl.BlockSpec((D, D), lambda i: (0, 0))
_b_spec = pl.BlockSpec((1, D), lambda i: (0, 0))

_half_out = jax.ShapeDtypeStruct((NPAD, HALF), jnp.float32)

_tc_pre = pl.pallas_call(
    _tc_pre_body,
    grid=(_GRID,),
    in_specs=[_row_spec, _w_spec, _deg_spec, _deg_spec],
    out_specs=[_half_spec, _half_spec],
    out_shape=[_half_out, _half_out],
)

_tc_mid = pl.pallas_call(
    _tc_mid_body,
    grid=(_GRID,),
    in_specs=[_half_spec, _half_spec, _half_spec, _half_spec,
              _deg_spec, _deg_spec, _w_spec, _b_spec],
    out_specs=[_half_spec, _half_spec],
    out_shape=[_half_out, _half_out],
)

_tc_fin = pl.pallas_call(
    _tc_fin_body,
    grid=(_GRID,),
    in_specs=[_half_spec, _half_spec, _half_spec, _half_spec,
              _deg_spec, _deg_spec, _w_spec, _b_spec, _b_spec],
    out_specs=_row_spec,
    out_shape=jax.ShapeDtypeStruct((NPAD, D), jnp.float32),
)


def kernel(x, edge_index, W1, b1, W2, b2, Wf, bf):
    src = edge_index[0]
    dst = edge_index[1]
    npad_e = EPAD - E
    src_t = jnp.concatenate(
        [src, jnp.zeros((npad_e,), jnp.int32)]).reshape(NS, CHUNKS, CHUNK)
    dst_p = jnp.concatenate([dst, jnp.full((npad_e,), TRASH, jnp.int32)])
    dst_t = dst_p.reshape(NS, CHUNKS, CHUNK)
    dst_d = dst_p.reshape(NC, NS, DCH, CHUNK)

    x_pad = jnp.concatenate([x, jnp.zeros((NPAD - N, D), jnp.float32)], axis=0)

    degp = _deg_kernel(dst_d)
    d0 = degp[0].reshape(NPAD, 1)
    d1 = degp[1].reshape(NPAD, 1)

    b1r = b1.reshape(1, D)
    b2r = b2.reshape(1, D)
    bfr = bf.reshape(1, D)

    hs1a, hs1b = _tc_pre(x_pad, W1, d0, d1)
    acc1 = _agg_kernel(hs1a, hs1b, src_t, dst_t)
    hs2a, hs2b = _tc_mid(acc1[0], acc1[1], hs1a, hs1b, d0, d1, W2, b1r)
    acc2 = _agg_kernel(hs2a, hs2b, src_t, dst_t)
    out = _tc_fin(acc2[0], acc2[1], hs2a, hs2b, d0, d1, Wf, b2r, bfr)
    return out[:N]


# speedup vs baseline: 5.3290x; 1.3433x over previous
import functools

import jax
import jax.numpy as jnp
from jax import lax
from jax.experimental import pallas as pl
from jax.experimental.pallas import tpu as pltpu
from jax.experimental.pallas import tpu_sc as plsc

N = 10000
D = 128
HALF = D // 2
E = 320000

NC = 2
NS = 16

CHUNK = 128
CHUNKS = 158
NB = 2
DCH = CHUNKS // 2
EPAD = NS * CHUNKS * CHUNK
TRASH = N
NPAD = 10240
ROWS_PER_TILE = NPAD // NS
RB = ROWS_PER_TILE // CHUNK

_mesh = plsc.VectorSubcoreMesh(core_axis_name="c", subcore_axis_name="s")


@functools.partial(
    pl.kernel,
    out_type=jax.ShapeDtypeStruct((NC, NPAD), jnp.float32),
    mesh=_mesh,
    scratch_types=[
        pltpu.VMEM((DCH, CHUNK), jnp.int32),
        pltpu.VMEM((CHUNK,), jnp.float32),
        pltpu.VMEM((ROWS_PER_TILE,), jnp.float32),
        pltpu.VMEM_SHARED((NPAD,), jnp.float32),
    ],
)
def _deg_kernel(dst_hbm, degp_hbm, dst_v, ones_v, dbuf_v, deg_sh):
    c = lax.axis_index("c")
    s = lax.axis_index("s")
    pltpu.sync_copy(dst_hbm.at[c, s], dst_v)

    @pl.loop(0, ROWS_PER_TILE // 16)
    def _zero(i):
        dbuf_v[pl.ds(i * 16, 16)] = jnp.zeros((16,), jnp.float32)

    for k in range(CHUNK // 16):
        ones_v[pl.ds(k * 16, 16)] = jnp.ones((16,), jnp.float32)
    pltpu.sync_copy(dbuf_v, deg_sh.at[pl.ds(s * ROWS_PER_TILE, ROWS_PER_TILE)])
    plsc.subcore_barrier()

    @pl.loop(0, DCH)
    def _acc(j):
        pltpu.sync_copy(ones_v, deg_sh.at[dst_v.at[j]], add=True)

    plsc.subcore_barrier()
    pltpu.sync_copy(deg_sh.at[pl.ds(s * ROWS_PER_TILE, ROWS_PER_TILE)], dbuf_v)
    pltpu.sync_copy(dbuf_v, degp_hbm.at[c, pl.ds(s * ROWS_PER_TILE, ROWS_PER_TILE)])


@functools.partial(
    pl.kernel,
    out_type=jax.ShapeDtypeStruct((NC, NPAD, HALF), jnp.float32),
    mesh=_mesh,
    compiler_params=pltpu.CompilerParams(use_tc_tiling_on_sc=False),
    scratch_types=[
        pltpu.VMEM((CHUNKS, CHUNK), jnp.int32),
        pltpu.VMEM((CHUNKS, CHUNK), jnp.int32),
        pltpu.VMEM((NB, CHUNK, HALF), jnp.float32),
        pltpu.VMEM_SHARED((NPAD, HALF), jnp.float32),
    ] + [pltpu.SemaphoreType.DMA] * NB,
)
def _agg_kernel(hs0_hbm, hs1_hbm, src_hbm, dst_hbm, accp_hbm,
                src_v, dst_v, rows_v, acc_sh, *sems):
    gs = sems
    c = lax.axis_index("c")
    s = lax.axis_index("s")
    pltpu.sync_copy(src_hbm.at[s], src_v)
    pltpu.sync_copy(dst_hbm.at[s], dst_v)

    @pl.loop(0, CHUNK)
    def _zero(r):
        for k in range(HALF // 16):
            rows_v[0, r, pl.ds(k * 16, 16)] = jnp.zeros((16,), jnp.float32)

    for t in range(RB):
        pltpu.sync_copy(
            rows_v.at[0], acc_sh.at[pl.ds(s * ROWS_PER_TILE + t * CHUNK, CHUNK)])
    plsc.subcore_barrier()

    def run(hs_hbm):
        pltpu.async_copy(hs_hbm.at[src_v.at[0]], rows_v.at[0], gs[0]).wait()

        @pl.loop(0, CHUNKS - 1)
        def _body(j):
            nxt = pltpu.async_copy(hs_hbm.at[src_v.at[j + 1]],
                                   rows_v.at[(j + 1) % 2], gs[1])
            pltpu.sync_copy(rows_v.at[j % 2], acc_sh.at[dst_v.at[j]], add=True)
            nxt.wait()

        pltpu.sync_copy(rows_v.at[(CHUNKS - 1) % 2],
                        acc_sh.at[dst_v.at[CHUNKS - 1]], add=True)

    @pl.when(c == 0)
    def _c0():
        run(hs0_hbm)

    @pl.when(c == 1)
    def _c1():
        run(hs1_hbm)

    plsc.subcore_barrier()
    for t in range(RB):
        row0 = s * ROWS_PER_TILE + t * CHUNK
        pltpu.sync_copy(acc_sh.at[pl.ds(row0, CHUNK)], rows_v.at[0])
        pltpu.sync_copy(rows_v.at[0], accp_hbm.at[c, pl.ds(row0, CHUNK)])


_BR = 512
_GRID = NPAD // _BR


def _tc_pre_body(x_ref, w_ref, d0_ref, d1_ref, hs0_ref, hs1_ref):
    dis = lax.rsqrt(d0_ref[...] + d1_ref[...] + 1.0)
    hs = jnp.dot(x_ref[...], w_ref[...],
                 preferred_element_type=jnp.float32) * dis
    hs0_ref[...] = hs[:, :HALF]
    hs1_ref[...] = hs[:, HALF:]


def _tc_mid_body(p0_ref, p1_ref, h0_ref, h1_ref, d0_ref, d1_ref, w_ref,
                 bi_ref, hs0_ref, hs1_ref):
    dis = lax.rsqrt(d0_ref[...] + d1_ref[...] + 1.0)
    a = jnp.concatenate(
        [p0_ref[...] + h0_ref[...],
         p1_ref[...] + h1_ref[...]],
        axis=1)
    a = jnp.maximum(dis * a + bi_ref[...], 0.0)
    hs = jnp.dot(a, w_ref[...], preferred_element_type=jnp.float32) * dis
    hs0_ref[...] = hs[:, :HALF]
    hs1_ref[...] = hs[:, HALF:]


def _tc_fin_body(p0_ref, p1_ref, h0_ref, h1_ref, d0_ref, d1_ref, w_ref,
                 bi_ref, bo_ref, out_ref):
    dis = lax.rsqrt(d0_ref[...] + d1_ref[...] + 1.0)
    a = jnp.concatenate(
        [p0_ref[...] + h0_ref[...],
         p1_ref[...] + h1_ref[...]],
        axis=1)
    a = jnp.maximum(dis * a + bi_ref[...], 0.0)
    out_ref[...] = jnp.dot(a, w_ref[...],
                           preferred_element_type=jnp.float32) + bo_ref[...]


_row_spec = pl.BlockSpec((_BR, D), lambda i: (i, 0))
_half_spec = pl.BlockSpec((_BR, HALF), lambda i: (i, 0))
_deg_spec = pl.BlockSpec((_BR, 1), lambda i: (i, 0))
_w_spec = pl.BlockSpec((D, D), lambda i: (0, 0))
_b_spec = pl.BlockSpec((1, D), lambda i: (0, 0))

_half_out = jax.ShapeDtypeStruct((NPAD, HALF), jnp.float32)

_tc_pre = pl.pallas_call(
    _tc_pre_body,
    grid=(_GRID,),
    in_specs=[_row_spec, _w_spec, _deg_spec, _deg_spec],
    out_specs=[_half_spec, _half_spec],
    out_shape=[_half_out, _half_out],
)

_tc_mid = pl.pallas_call(
    _tc_mid_body,
    grid=(_GRID,),
    in_specs=[_half_spec, _half_spec, _half_spec, _half_spec,
              _deg_spec, _deg_spec, _w_spec, _b_spec],
    out_specs=[_half_spec, _half_spec],
    out_shape=[_half_out, _half_out],
)

_tc_fin = pl.pallas_call(
    _tc_fin_body,
    grid=(_GRID,),
    in_specs=[_half_spec, _half_spec, _half_spec, _half_spec,
              _deg_spec, _deg_spec, _w_spec, _b_spec, _b_spec],
    out_specs=_row_spec,
    out_shape=jax.ShapeDtypeStruct((NPAD, D), jnp.float32),
)


def kernel(x, edge_index, W1, b1, W2, b2, Wf, bf):
    src = edge_index[0]
    dst = edge_index[1]
    npad_e = EPAD - E
    src_t = jnp.concatenate(
        [src, jnp.zeros((npad_e,), jnp.int32)]).reshape(NS, CHUNKS, CHUNK)
    dst_p = jnp.concatenate([dst, jnp.full((npad_e,), TRASH, jnp.int32)])
    dst_t = dst_p.reshape(NS, CHUNKS, CHUNK)
    dst_d = dst_p.reshape(NC, NS, DCH, CHUNK)

    x_pad = jnp.concatenate([x, jnp.zeros((NPAD - N, D), jnp.float32)], axis=0)

    degp = _deg_kernel(dst_d)
    d0 = degp[0].reshape(NPAD, 1)
    d1 = degp[1].reshape(NPAD, 1)

    b1r = b1.reshape(1, D)
    b2r = b2.reshape(1, D)
    bfr = bf.reshape(1, D)

    hs1a, hs1b = _tc_pre(x_pad, W1, d0, d1)
    acc1 = _agg_kernel(hs1a, hs1b, src_t, dst_t)
    hs2a, hs2b = _tc_mid(acc1[0], acc1[1], hs1a, hs1b, d0, d1, W2, b1r)
    acc2 = _agg_kernel(hs2a, hs2b, src_t, dst_t)
    out = _tc_fin(acc2[0], acc2[1], hs2a, hs2b, d0, d1, Wf, b2r, bfr)
    return out[:N]
